# Initial kernel scaffold; baseline (speedup 1.0000x reference)
import jax
import jax.numpy as jnp
from jax.experimental import pallas as pl


def kernel(x, edge_index, edge_attr, batch, W_sim1, b_sim1, W_lin1, b_lin1, W_e1a, b_e1a, W_e1b, b_e1b, W_sim2, b_sim2, W_lin2, b_lin2, W_e2a, b_e2a, W_e2b, b_e2b, W_fc, b_fc):
    raise NotImplementedError("write your pallas kernel here")


# speedup vs baseline: 5.7437x; 5.7437x over previous
import functools

import jax
import jax.numpy as jnp
import numpy as np
from jax.experimental import pallas as pl
from jax.experimental.pallas import tpu as pltpu

N_NODES_C = 10000
N_EDGES_C = 320000
NUM_KERNELS_C = 4
NUM_GRAPHS_C = 16
KM_ITERS = 300

_A = float(jax.nn.softplus(1.0))

_IDX0_1 = tuple(int(v) for v in np.asarray(
    jax.random.randint(jax.random.key(1), (NUM_KERNELS_C,), 0, N_NODES_C)))
_IDX0_2 = tuple(int(v) for v in np.asarray(
    jax.random.randint(jax.random.key(2), (NUM_KERNELS_C,), 0, N_NODES_C)))


def _prologue_body(x_ref, Wsim_ref, bsim_ref, Wlin_ref, blin_ref, WP_ref,
                   bea_ref, WQ_ref, xl_ref, P_ref, Q_ref):
    x = x_ref[...]
    sim = jax.nn.relu(jnp.dot(x, Wsim_ref[...]) + bsim_ref[...])
    xl_ref[...] = jax.nn.relu(jnp.dot(x, Wlin_ref[...]) + blin_ref[...])
    P_ref[...] = jnp.dot(sim, WP_ref[...]) + bea_ref[...]
    Q_ref[...] = jnp.dot(sim, WQ_ref[...])


def _prologue(x, W_sim, b_sim, W_lin, b_lin, W_ea, b_ea):
    n, fin = x.shape
    fsim = W_sim.shape[1]
    fxl = W_lin.shape[1]
    g = W_ea.shape[0] // 2
    Wsim_eff = W_sim[:fin] + W_sim[fin:]
    WP = W_ea[:g] - W_ea[g:]
    WQ = W_ea[g:]
    return pl.pallas_call(
        _prologue_body,
        out_shape=[
            jax.ShapeDtypeStruct((n, fxl), jnp.float32),
            jax.ShapeDtypeStruct((n, 64), jnp.float32),
            jax.ShapeDtypeStruct((n, 64), jnp.float32),
        ],
    )(x, Wsim_eff, b_sim.reshape(1, fsim), W_lin, b_lin.reshape(1, fxl),
      WP, b_ea.reshape(1, 64), WQ)


def _edge_mm_body(Gd_ref, Gs_ref, W_ref, R_ref):
    t = jax.nn.relu(Gd_ref[...] + Gs_ref[...])
    R_ref[...] = jnp.dot(t, W_ref[...])


def _edge_mm(Gd, Gs, W_eb, block=8000):
    e = Gd.shape[0]
    grid = e // block
    return pl.pallas_call(
        _edge_mm_body,
        grid=(grid,),
        in_specs=[
            pl.BlockSpec((block, 64), lambda i: (i, 0)),
            pl.BlockSpec((block, 64), lambda i: (i, 0)),
            pl.BlockSpec((64, 64), lambda i: (0, 0)),
        ],
        out_specs=pl.BlockSpec((block, 64), lambda i: (i, 0)),
        out_shape=jax.ShapeDtypeStruct((e, 64), jnp.float32),
    )(Gd, Gs, W_eb)


def _kmeans_body(idx0, scores_ref, assign_ref, prev_ref):
    X = scores_ref[...]
    cent0 = jnp.concatenate([X[i:i + 1] for i in idx0], axis=0)
    prev_ref[...] = jnp.full((N_NODES_C, 1), -1, jnp.int32)

    def dists_assign(cent):
        dmin = None
        assign = None
        for k in range(NUM_KERNELS_C):
            ck = cent[k:k + 1]
            dk = jnp.sqrt(jnp.sum((X - ck) ** 2, axis=1, keepdims=True))
            if k == 0:
                dmin = dk
                assign = jnp.zeros((N_NODES_C, 1), jnp.int32)
            else:
                lt = dk < dmin
                assign = jnp.where(lt, k, assign)
                dmin = jnp.where(lt, dk, dmin)
        return assign

    def cond(carry):
        it, changed, _ = carry
        return (it < KM_ITERS) & changed

    def body(carry):
        it, _, cent = carry
        assign = dists_assign(cent)
        changed = jnp.any(assign != prev_ref[...])
        prev_ref[...] = assign
        rows = []
        for k in range(NUM_KERNELS_C):
            mask = (assign == k).astype(jnp.float32)
            s = jnp.sum(X * mask, axis=0, keepdims=True)
            c = jnp.sum(mask)
            rows.append(s / jnp.maximum(c, 1.0))
        return it + 1, changed, jnp.concatenate(rows, axis=0)

    _, _, cent = jax.lax.while_loop(cond, body, (0, True, cent0))
    assign_ref[...] = dists_assign(cent)


def _kmeans(scores, idx0):
    out = pl.pallas_call(
        functools.partial(_kmeans_body, idx0),
        out_shape=jax.ShapeDtypeStruct((N_NODES_C, 1), jnp.int32),
        scratch_shapes=[pltpu.VMEM((N_NODES_C, 1), jnp.int32)],
    )(scores)
    return out.reshape(N_NODES_C)


def _final_body(xl_ref, M_ref, batch_ref, Wfc_ref, bfc_ref, out_ref):
    M = M_ref[...]
    has = ~jnp.isneginf(M)
    h2 = jnp.where(has, xl_ref[...] * jnp.where(has, M + 1e-5, 0.0), 0.0)
    onehot = (batch_ref[...] ==
              jax.lax.broadcasted_iota(jnp.int32, (N_NODES_C, NUM_GRAPHS_C), 1)
              ).astype(jnp.float32)
    sums = jax.lax.dot_general(onehot, h2, (((0,), (0,)), ((), ())))
    ones = jnp.ones((N_NODES_C, 1), jnp.float32)
    cnts = jax.lax.dot_general(onehot, ones, (((0,), (0,)), ((), ())))
    pooled = sums / jnp.maximum(cnts, 1.0)
    out_ref[...] = jnp.dot(pooled, Wfc_ref[...]) + bfc_ref[...]


def _final(xl2, M2, batch, W_fc, b_fc):
    return pl.pallas_call(
        _final_body,
        out_shape=jax.ShapeDtypeStruct((NUM_GRAPHS_C, 1), jnp.float32),
    )(xl2, M2.reshape(N_NODES_C, 1), batch.reshape(N_NODES_C, 1),
      W_fc, b_fc.reshape(1, 1))


def _layer(x, src, dst, deg, w_e, W_sim, b_sim, W_lin, b_lin, W_ea, b_ea,
           W_eb, b_eb, idx0):
    n = x.shape[0]
    xl, P, Q = _prologue(x, W_sim, b_sim, W_lin, b_lin, W_ea, b_ea)
    Gd = P[dst]
    Gs = Q[src]
    R = _edge_mm(Gd, Gs, W_eb)
    mx = jax.ops.segment_max(R, dst, num_segments=n)
    has = ~jnp.isneginf(mx[:, 0])
    scores = jnp.where(has[:, None], mx + b_eb, 0.0)
    assign = _kmeans(scores, idx0)
    same = assign[src] == assign[dst]
    val = jnp.where(same, w_e / deg[src], 0.0)
    M = jax.ops.segment_max(val, dst, num_segments=n)
    return xl, M


def kernel(x, edge_index, edge_attr, batch, W_sim1, b_sim1, W_lin1, b_lin1,
           W_e1a, b_e1a, W_e1b, b_e1b, W_sim2, b_sim2, W_lin2, b_lin2,
           W_e2a, b_e2a, W_e2b, b_e2b, W_fc, b_fc):
    src = edge_index[0]
    dst = edge_index[1]
    n = x.shape[0]
    deg = jnp.zeros((n,), jnp.float32).at[src].add(1.0)
    w_e = _A * edge_attr ** _A

    xl1, M1 = _layer(x, src, dst, deg, w_e, W_sim1, b_sim1, W_lin1, b_lin1,
                     W_e1a, b_e1a, W_e1b, b_e1b, _IDX0_1)
    has1 = ~jnp.isneginf(M1)
    x2 = jnp.where(has1[:, None], xl1 * jnp.where(has1, M1 + 1e-5, 0.0)[:, None], 0.0)

    xl2, M2 = _layer(x2, src, dst, deg, w_e, W_sim2, b_sim2, W_lin2, b_lin2,
                     W_e2a, b_e2a, W_e2b, b_e2b, _IDX0_2)
    return _final(xl2, M2, batch, W_fc, b_fc)


# speedup vs baseline: 6.5820x; 1.1460x over previous
import functools

import jax
import jax.numpy as jnp
import numpy as np
from jax import lax
from jax.experimental import pallas as pl
from jax.experimental.pallas import tpu as pltpu
from jax.experimental.pallas import tpu_sc as plsc


N_NODES_C = 10000
N_EDGES_C = 320000
NUM_KERNELS_C = 4
NUM_GRAPHS_C = 16
KM_ITERS = 300

_SC_CORES = 2
_SC_SUBCORES = 16
_NW = _SC_CORES * _SC_SUBCORES
_GROW = 128
_NROWS = N_EDGES_C // _GROW
_ROWS_PER_W = _NROWS // _NW
_ROWS_REM = _NROWS - _ROWS_PER_W * _NW
_SC_MESH = plsc.VectorSubcoreMesh(
    core_axis_name="c", subcore_axis_name="s",
    num_cores=_SC_CORES, num_subcores=_SC_SUBCORES)

_IDX0_1 = (7996, 2927, 3040, 1353)
_IDX0_2 = (7999, 6492, 7913, 1503)


def _prologue_body(x_ref, Wsim_ref, bsim_ref, Wlin_ref, blin_ref, WPQ_ref,
                   bea_ref, xl_ref, T_ref):
    x = x_ref[...]
    sim = jax.nn.relu(jnp.dot(x, Wsim_ref[...]) + bsim_ref[...])
    xl_ref[...] = jax.nn.relu(jnp.dot(x, Wlin_ref[...]) + blin_ref[...])
    T_ref[...] = jnp.dot(sim, WPQ_ref[...]) + bea_ref[...]


def _prologue(x, W_sim, b_sim, W_lin, b_lin, W_ea, b_ea):
    n, fin = x.shape
    fsim = W_sim.shape[1]
    fxl = W_lin.shape[1]
    g = W_ea.shape[0] // 2
    Wsim_eff = W_sim[:fin] + W_sim[fin:]
    WPQ = jnp.concatenate([W_ea[:g] - W_ea[g:], W_ea[g:]], axis=1)
    bea_pad = jnp.concatenate([b_ea, jnp.zeros((64,), jnp.float32)])
    return pl.pallas_call(
        _prologue_body,
        out_shape=[
            jax.ShapeDtypeStruct((n, fxl), jnp.float32),
            jax.ShapeDtypeStruct((n, 128), jnp.float32),
        ],
    )(x, Wsim_eff, b_sim.reshape(1, fsim), W_lin, b_lin.reshape(1, fxl),
      WPQ, bea_pad.reshape(1, 128))


def _sc_gather_body(T, d2, s2, U,
                    idxd0, idxd1, idxs0, idxs1, bD0, bD1, bS0, bS1,
                    sp0, sp1, sq0, sq1):
    wid = lax.axis_index("s") * _SC_CORES + lax.axis_index("c")
    idxd = (idxd0, idxd1)
    idxs = (idxs0, idxs1)
    bD = (bD0, bD1)
    bS = (bS0, bS1)
    sp = (sp0, sp1)
    sq = (sq0, sq1)

    def fire(j, b):
        r = wid + j * _NW
        pltpu.sync_copy(d2.at[r], idxd[b])
        pltpu.async_copy(T.at[idxd[b]], bD[b], sp[b])
        pltpu.sync_copy(s2.at[r], idxs[b])
        pltpu.async_copy(T.at[idxs[b]], bS[b], sq[b])

    def drain(j, b):
        r = wid + j * _NW
        pltpu.make_async_copy(T.at[idxd[b]], bD[b], sp[b]).wait()
        pltpu.make_async_copy(T.at[idxs[b]], bS[b], sq[b]).wait()

        def row(i, _):
            for c in range(4):
                u = bD[b][i, pl.ds(c * 16, 16)] + bS[b][i, pl.ds(64 + c * 16, 16)]
                bD[b][i, pl.ds(c * 16, 16)] = jnp.maximum(u, 0.0)
            return 0

        lax.fori_loop(0, _GROW, row, 0)
        pltpu.sync_copy(bD[b], U.at[pl.ds(r * _GROW, _GROW)])

    fire(0, 0)

    def lbody(i, _):
        g = 2 * i
        fire(g + 1, 1)
        drain(g, 0)

        @pl.when(g + 2 < _ROWS_PER_W)
        def _fire_next():
            fire(g + 2, 0)
        drain(g + 1, 1)
        return 0

    lax.fori_loop(0, _ROWS_PER_W // 2, lbody, 0)

    @pl.when(wid < _ROWS_REM)
    def _tail():
        fire(_ROWS_PER_W, 0)
        drain(_ROWS_PER_W, 0)


def _sc_gather(T, dst, src):
    e = dst.shape[0]
    d2 = dst.reshape(_NROWS, _GROW)
    s2 = src.reshape(_NROWS, _GROW)
    return pl.kernel(
        _sc_gather_body,
        out_type=jax.ShapeDtypeStruct((e, 128), jnp.float32),
        mesh=_SC_MESH,
        scratch_types=[
            pltpu.VMEM((_GROW,), jnp.int32), pltpu.VMEM((_GROW,), jnp.int32),
            pltpu.VMEM((_GROW,), jnp.int32), pltpu.VMEM((_GROW,), jnp.int32),
            pltpu.VMEM((_GROW, 128), jnp.float32), pltpu.VMEM((_GROW, 128), jnp.float32),
            pltpu.VMEM((_GROW, 128), jnp.float32), pltpu.VMEM((_GROW, 128), jnp.float32),
            pltpu.SemaphoreType.DMA, pltpu.SemaphoreType.DMA,
            pltpu.SemaphoreType.DMA, pltpu.SemaphoreType.DMA,
        ],
    )(T, d2, s2)


def _edge_mm_body(U_ref, W_ref, R_ref):
    R_ref[...] = jnp.dot(U_ref[...], W_ref[...])


def _edge_mm(U, W_eb, block=8000):
    e = U.shape[0]
    grid = e // block
    Wpad = jnp.concatenate([W_eb, jnp.zeros((64, 64), jnp.float32)], axis=0)
    return pl.pallas_call(
        _edge_mm_body,
        grid=(grid,),
        in_specs=[
            pl.BlockSpec((block, 128), lambda i: (i, 0)),
            pl.BlockSpec((128, 64), lambda i: (0, 0)),
        ],
        out_specs=pl.BlockSpec((block, 64), lambda i: (i, 0)),
        out_shape=jax.ShapeDtypeStruct((e, 64), jnp.float32),
    )(U, Wpad)


def _kmeans_body(idx0, scores_ref, assign_ref, prev_ref):
    X = scores_ref[...]
    cent0 = jnp.concatenate([X[i:i + 1] for i in idx0], axis=0)
    prev_ref[...] = jnp.full((N_NODES_C, 1), -1, jnp.int32)

    def dists_assign(cent):
        dmin = None
        assign = None
        for k in range(NUM_KERNELS_C):
            ck = cent[k:k + 1]
            dk = jnp.sqrt(jnp.sum((X - ck) ** 2, axis=1, keepdims=True))
            if k == 0:
                dmin = dk
                assign = jnp.zeros((N_NODES_C, 1), jnp.int32)
            else:
                lt = dk < dmin
                assign = jnp.where(lt, k, assign)
                dmin = jnp.where(lt, dk, dmin)
        return assign

    def cond(carry):
        it, changed, _ = carry
        return (it < KM_ITERS) & changed

    def body(carry):
        it, _, cent = carry
        assign = dists_assign(cent)
        changed = jnp.any(assign != prev_ref[...])
        prev_ref[...] = assign
        rows = []
        for k in range(NUM_KERNELS_C):
            mask = (assign == k).astype(jnp.float32)
            s = jnp.sum(X * mask, axis=0, keepdims=True)
            c = jnp.sum(mask)
            rows.append(s / jnp.maximum(c, 1.0))
        return it + 1, changed, jnp.concatenate(rows, axis=0)

    _, _, cent = jax.lax.while_loop(cond, body, (0, True, cent0))
    assign_ref[...] = dists_assign(cent)


def _kmeans(scores, idx0):
    out = pl.pallas_call(
        functools.partial(_kmeans_body, idx0),
        out_shape=jax.ShapeDtypeStruct((N_NODES_C, 1), jnp.int32),
        scratch_shapes=[pltpu.VMEM((N_NODES_C, 1), jnp.int32)],
    )(scores)
    return out.reshape(N_NODES_C)


def _final_body(xl_ref, M_ref, batch_ref, Wfc_ref, bfc_ref, out_ref):
    M = M_ref[...]
    has = ~jnp.isneginf(M)
    h2 = jnp.where(has, xl_ref[...] * jnp.where(has, M + 1e-5, 0.0), 0.0)
    onehot = (batch_ref[...] ==
              jax.lax.broadcasted_iota(jnp.int32, (N_NODES_C, NUM_GRAPHS_C), 1)
              ).astype(jnp.float32)
    sums = jax.lax.dot_general(onehot, h2, (((0,), (0,)), ((), ())))
    ones = jnp.ones((N_NODES_C, 1), jnp.float32)
    cnts = jax.lax.dot_general(onehot, ones, (((0,), (0,)), ((), ())))
    pooled = sums / jnp.maximum(cnts, 1.0)
    out_ref[...] = jnp.dot(pooled, Wfc_ref[...]) + bfc_ref[...]


def _final(xl2, M2, batch, W_fc, b_fc):
    return pl.pallas_call(
        _final_body,
        out_shape=jax.ShapeDtypeStruct((NUM_GRAPHS_C, 1), jnp.float32),
    )(xl2, M2.reshape(N_NODES_C, 1), batch.reshape(N_NODES_C, 1),
      W_fc, b_fc.reshape(1, 1))


def _layer(x, src, dst, deg, w_e, W_sim, b_sim, W_lin, b_lin, W_ea, b_ea,
           W_eb, b_eb, idx0):
    n = x.shape[0]
    xl, T = _prologue(x, W_sim, b_sim, W_lin, b_lin, W_ea, b_ea)
    U = _sc_gather(T, dst, src)
    R = _edge_mm(U, W_eb)
    mx = jax.ops.segment_max(R, dst, num_segments=n)
    has = ~jnp.isneginf(mx[:, 0])
    scores = jnp.where(has[:, None], mx + b_eb, 0.0)
    assign = _kmeans(scores, idx0)
    same = assign[src] == assign[dst]
    val = jnp.where(same, w_e / deg[src], 0.0)
    M = jax.ops.segment_max(val, dst, num_segments=n)
    return xl, M


def kernel(x, edge_index, edge_attr, batch, W_sim1, b_sim1, W_lin1, b_lin1,
           W_e1a, b_e1a, W_e1b, b_e1b, W_sim2, b_sim2, W_lin2, b_lin2,
           W_e2a, b_e2a, W_e2b, b_e2b, W_fc, b_fc):
    src = edge_index[0]
    dst = edge_index[1]
    n = x.shape[0]
    deg = jnp.zeros((n,), jnp.float32).at[src].add(1.0)
    a_c = jax.nn.softplus(jnp.float32(1.0))
    w_e = a_c * edge_attr ** a_c

    xl1, M1 = _layer(x, src, dst, deg, w_e, W_sim1, b_sim1, W_lin1, b_lin1,
                     W_e1a, b_e1a, W_e1b, b_e1b, _IDX0_1)
    has1 = ~jnp.isneginf(M1)
    x2 = jnp.where(has1[:, None], xl1 * jnp.where(has1, M1 + 1e-5, 0.0)[:, None], 0.0)

    xl2, M2 = _layer(x2, src, dst, deg, w_e, W_sim2, b_sim2, W_lin2, b_lin2,
                     W_e2a, b_e2a, W_e2b, b_e2b, _IDX0_2)
    return _final(xl2, M2, batch, W_fc, b_fc)


# speedup vs baseline: 27.1254x; 4.1211x over previous
import functools

import jax
import jax.numpy as jnp
import numpy as np
from jax import lax
from jax.experimental import pallas as pl
from jax.experimental.pallas import tpu as pltpu
from jax.experimental.pallas import tpu_sc as plsc


N_NODES_C = 10000
N_EDGES_C = 320000
NUM_KERNELS_C = 4
NUM_GRAPHS_C = 16
KM_ITERS = 300

_SC_CORES = 2
_SC_SUBCORES = 16
_NW = _SC_CORES * _SC_SUBCORES
_GROW = 128
_NROWS = N_EDGES_C // _GROW
_ROWS_PER_W = _NROWS // _NW
_ROWS_REM = _NROWS - _ROWS_PER_W * _NW
_MROWS_PER_W = 80
_MROWS = _MROWS_PER_W * _NW
_SC_MESH = plsc.VectorSubcoreMesh(
    core_axis_name="c", subcore_axis_name="s",
    num_cores=_SC_CORES, num_subcores=_SC_SUBCORES)

_IDX0_1 = (7996, 2927, 3040, 1353)
_IDX0_2 = (7999, 6492, 7913, 1503)


def _prologue_body(x_ref, Wsim_ref, bsim_ref, Wlin_ref, blin_ref, WPQ_ref,
                   bea_ref, xl_ref, T_ref):
    x = x_ref[...]
    sim = jax.nn.relu(jnp.dot(x, Wsim_ref[...]) + bsim_ref[...])
    xl_ref[...] = jax.nn.relu(jnp.dot(x, Wlin_ref[...]) + blin_ref[...])
    T_ref[...] = jnp.dot(sim, WPQ_ref[...]) + bea_ref[...]


def _prologue2_body(xlp_ref, Mcol_ref, Wsim_ref, bsim_ref, Wlin_ref, blin_ref,
                    WPQ_ref, bea_ref, xl_ref, T_ref):
    Mred = jnp.max(Mcol_ref[...], axis=1, keepdims=True)
    has = Mred >= 0.0
    x = jnp.where(has, xlp_ref[...] * jnp.where(has, Mred + 1e-5, 0.0), 0.0)
    sim = jax.nn.relu(jnp.dot(x, Wsim_ref[...]) + bsim_ref[...])
    xl_ref[...] = jax.nn.relu(jnp.dot(x, Wlin_ref[...]) + blin_ref[...])
    T_ref[...] = jnp.dot(sim, WPQ_ref[...]) + bea_ref[...]


def _prologue(x_or_parts, W_sim, b_sim, W_lin, b_lin, W_ea, b_ea):
    fsim = W_sim.shape[1]
    fin = W_sim.shape[0] // 2
    fxl = W_lin.shape[1]
    g = W_ea.shape[0] // 2
    n = N_NODES_C
    Wsim_eff = W_sim[:fin] + W_sim[fin:]
    WPQ = jnp.concatenate([W_ea[:g] - W_ea[g:], W_ea[g:]], axis=1)
    bea_pad = jnp.concatenate([b_ea, jnp.zeros((64,), jnp.float32)])
    weights = (Wsim_eff, b_sim.reshape(1, fsim), W_lin, b_lin.reshape(1, fxl),
               WPQ, bea_pad.reshape(1, 128))
    out_shape = [
        jax.ShapeDtypeStruct((n, fxl), jnp.float32),
        jax.ShapeDtypeStruct((n, 128), jnp.float32),
    ]
    if isinstance(x_or_parts, tuple):
        xl_prev, Mcol_prev = x_or_parts
        return pl.pallas_call(_prologue2_body, out_shape=out_shape)(
            xl_prev, Mcol_prev, *weights)
    return pl.pallas_call(_prologue_body, out_shape=out_shape)(
        x_or_parts, *weights)


def _sc_gather_body(T, d2, s2, U,
                    idxd0, idxd1, idxs0, idxs1, bD0, bD1, bS0, bS1,
                    sp0, sp1, sq0, sq1):
    wid = lax.axis_index("s") * _SC_CORES + lax.axis_index("c")
    idxd = (idxd0, idxd1)
    idxs = (idxs0, idxs1)
    bD = (bD0, bD1)
    bS = (bS0, bS1)
    sp = (sp0, sp1)
    sq = (sq0, sq1)

    def fire(j, b):
        r = wid + j * _NW
        pltpu.sync_copy(d2.at[r], idxd[b])
        pltpu.async_copy(T.at[idxd[b]], bD[b], sp[b])
        pltpu.sync_copy(s2.at[r], idxs[b])
        pltpu.async_copy(T.at[idxs[b]], bS[b], sq[b])

    def drain(j, b):
        r = wid + j * _NW
        pltpu.make_async_copy(T.at[idxd[b]], bD[b], sp[b]).wait()
        pltpu.make_async_copy(T.at[idxs[b]], bS[b], sq[b]).wait()

        def row(i, _):
            for c in range(4):
                u = bD[b][i, pl.ds(c * 16, 16)] + bS[b][i, pl.ds(64 + c * 16, 16)]
                bD[b][i, pl.ds(c * 16, 16)] = jnp.maximum(u, 0.0)
            return 0

        lax.fori_loop(0, _GROW, row, 0)
        pltpu.sync_copy(bD[b], U.at[pl.ds(r * _GROW, _GROW)])

    fire(0, 0)

    def lbody(i, _):
        g = 2 * i
        fire(g + 1, 1)
        drain(g, 0)

        @pl.when(g + 2 < _ROWS_PER_W)
        def _fire_next():
            fire(g + 2, 0)
        drain(g + 1, 1)
        return 0

    lax.fori_loop(0, _ROWS_PER_W // 2, lbody, 0)

    @pl.when(wid < _ROWS_REM)
    def _tail():
        fire(_ROWS_PER_W, 0)
        drain(_ROWS_PER_W, 0)


def _sc_gather(T, dst, src):
    e = dst.shape[0]
    d2 = dst.reshape(_NROWS, _GROW)
    s2 = src.reshape(_NROWS, _GROW)
    return pl.kernel(
        _sc_gather_body,
        out_type=jax.ShapeDtypeStruct((e, 128), jnp.float32),
        mesh=_SC_MESH,
        scratch_types=[
            pltpu.VMEM((_GROW,), jnp.int32), pltpu.VMEM((_GROW,), jnp.int32),
            pltpu.VMEM((_GROW,), jnp.int32), pltpu.VMEM((_GROW,), jnp.int32),
            pltpu.VMEM((_GROW, 128), jnp.float32), pltpu.VMEM((_GROW, 128), jnp.float32),
            pltpu.VMEM((_GROW, 128), jnp.float32), pltpu.VMEM((_GROW, 128), jnp.float32),
            pltpu.SemaphoreType.DMA, pltpu.SemaphoreType.DMA,
            pltpu.SemaphoreType.DMA, pltpu.SemaphoreType.DMA,
        ],
    )(T, d2, s2)


def _sc_mask_body(src2, dst2, w2, assign_hbm, deg_hbm, Mpart,
                  sbuf, dbuf, wbuf, abuf, gbuf, macc):
    wid = lax.axis_index("s") * _SC_CORES + lax.axis_index("c")
    base = wid * _MROWS_PER_W

    pltpu.sync_copy(assign_hbm, abuf)
    pltpu.sync_copy(deg_hbm, gbuf)
    pltpu.sync_copy(src2.at[pl.ds(base, _MROWS_PER_W)], sbuf)
    pltpu.sync_copy(dst2.at[pl.ds(base, _MROWS_PER_W)], dbuf)
    pltpu.sync_copy(w2.at[pl.ds(base, _MROWS_PER_W)], wbuf)

    def init(i, _):
        macc[pl.ds(i * 16, 16)] = jnp.full((16,), -1.0, jnp.float32)
        return 0

    lax.fori_loop(0, N_NODES_C // 16, init, 0)

    def row(j, _):
        for v in range(_GROW // 16):
            sl = pl.ds(v * 16, 16)
            s_idx = sbuf[j, sl]
            d_idx = dbuf[j, sl]
            w = wbuf[j, sl]
            a_s = plsc.load_gather(abuf, [s_idx])
            a_d = plsc.load_gather(abuf, [d_idx])
            dg = plsc.load_gather(gbuf, [s_idx])
            val = jnp.where(a_s == a_d, w / dg, 0.0)

            def cond(pending):
                return jnp.any(pending)

            def step(pending):
                cur = plsc.load_gather(macc, [d_idx])
                need = pending & (val > cur)
                plsc.store_scatter(macc, [d_idx], val, mask=need)
                chk = plsc.load_gather(macc, [d_idx])
                return need & (chk < val)

            lax.while_loop(cond, step,
                           jnp.full((16,), True, jnp.bool_))
        return 0

    lax.fori_loop(0, _MROWS_PER_W, row, 0)
    pltpu.sync_copy(macc, Mpart.at[wid])


def _sc_mask_max(src2, dst2, w2, assign, deg):
    return pl.kernel(
        _sc_mask_body,
        out_type=jax.ShapeDtypeStruct((_NW, N_NODES_C), jnp.float32),
        mesh=_SC_MESH,
        compiler_params=pltpu.CompilerParams(needs_layout_passes=False),
        scratch_types=[
            pltpu.VMEM((_MROWS_PER_W, _GROW), jnp.int32),
            pltpu.VMEM((_MROWS_PER_W, _GROW), jnp.int32),
            pltpu.VMEM((_MROWS_PER_W, _GROW), jnp.float32),
            pltpu.VMEM((N_NODES_C,), jnp.int32),
            pltpu.VMEM((N_NODES_C,), jnp.float32),
            pltpu.VMEM((N_NODES_C,), jnp.float32),
        ],
    )(src2, dst2, w2, assign, deg)


def _edge_mm_body(U_ref, W_ref, R_ref):
    R_ref[...] = jnp.dot(U_ref[...], W_ref[...])


def _edge_mm(U, W_eb, block=8000):
    e = U.shape[0]
    grid = e // block
    Wpad = jnp.concatenate([W_eb, jnp.zeros((64, 64), jnp.float32)], axis=0)
    return pl.pallas_call(
        _edge_mm_body,
        grid=(grid,),
        in_specs=[
            pl.BlockSpec((block, 128), lambda i: (i, 0)),
            pl.BlockSpec((128, 64), lambda i: (0, 0)),
        ],
        out_specs=pl.BlockSpec((block, 64), lambda i: (i, 0)),
        out_shape=jax.ShapeDtypeStruct((e, 64), jnp.float32),
    )(U, Wpad)


def _kmeans_body(idx0, scores_ref, assign_ref, prev_ref):
    X = scores_ref[...]
    cent0 = jnp.concatenate([X[i:i + 1] for i in idx0], axis=0)
    prev_ref[...] = jnp.full((N_NODES_C, 1), -1, jnp.int32)

    def dists_assign(cent):
        dmin = None
        assign = None
        for k in range(NUM_KERNELS_C):
            ck = cent[k:k + 1]
            dk = jnp.sqrt(jnp.sum((X - ck) ** 2, axis=1, keepdims=True))
            if k == 0:
                dmin = dk
                assign = jnp.zeros((N_NODES_C, 1), jnp.int32)
            else:
                lt = dk < dmin
                assign = jnp.where(lt, k, assign)
                dmin = jnp.where(lt, dk, dmin)
        return assign

    def cond(carry):
        it, changed, _ = carry
        return (it < KM_ITERS) & changed

    def body(carry):
        it, _, cent = carry
        assign = dists_assign(cent)
        changed = jnp.any(assign != prev_ref[...])
        prev_ref[...] = assign
        rows = []
        for k in range(NUM_KERNELS_C):
            mask = (assign == k).astype(jnp.float32)
            s = jnp.sum(X * mask, axis=0, keepdims=True)
            c = jnp.sum(mask)
            rows.append(s / jnp.maximum(c, 1.0))
        return it + 1, changed, jnp.concatenate(rows, axis=0)

    _, _, cent = jax.lax.while_loop(cond, body, (0, True, cent0))
    assign_ref[...] = dists_assign(cent)


def _kmeans(scores, idx0):
    out = pl.pallas_call(
        functools.partial(_kmeans_body, idx0),
        out_shape=jax.ShapeDtypeStruct((N_NODES_C, 1), jnp.int32),
        scratch_shapes=[pltpu.VMEM((N_NODES_C, 1), jnp.int32)],
    )(scores)
    return out.reshape(N_NODES_C)


def _final_body(xl_ref, Mcol_ref, batch_ref, Wfc_ref, bfc_ref, out_ref):
    Mred = jnp.max(Mcol_ref[...], axis=1, keepdims=True)
    has = Mred >= 0.0
    h2 = jnp.where(has, xl_ref[...] * jnp.where(has, Mred + 1e-5, 0.0), 0.0)
    onehot = (batch_ref[...] ==
              jax.lax.broadcasted_iota(jnp.int32, (N_NODES_C, NUM_GRAPHS_C), 1)
              ).astype(jnp.float32)
    sums = jax.lax.dot_general(onehot, h2, (((0,), (0,)), ((), ())))
    ones = jnp.ones((N_NODES_C, 1), jnp.float32)
    cnts = jax.lax.dot_general(onehot, ones, (((0,), (0,)), ((), ())))
    pooled = sums / jnp.maximum(cnts, 1.0)
    out_ref[...] = jnp.dot(pooled, Wfc_ref[...]) + bfc_ref[...]


def _final(xl2, Mcol2, batch, W_fc, b_fc):
    return pl.pallas_call(
        _final_body,
        out_shape=jax.ShapeDtypeStruct((NUM_GRAPHS_C, 1), jnp.float32),
    )(xl2, Mcol2, batch.reshape(N_NODES_C, 1), W_fc, b_fc.reshape(1, 1))


def _layer(x_or_parts, src, dst, edge2, W_sim, b_sim, W_lin, b_lin, W_ea,
           b_ea, W_eb, b_eb, idx0):
    n = N_NODES_C
    src2p, dst2p, w2p, deg = edge2
    xl, T = _prologue(x_or_parts, W_sim, b_sim, W_lin, b_lin, W_ea, b_ea)
    U = _sc_gather(T, dst, src)
    R = _edge_mm(U, W_eb)
    mx = jax.ops.segment_max(R, dst, num_segments=n)
    has = ~jnp.isneginf(mx[:, 0])
    scores = jnp.where(has[:, None], mx + b_eb, 0.0)
    assign = _kmeans(scores, idx0)
    Mpart = _sc_mask_max(src2p, dst2p, w2p, assign, deg)
    return xl, jnp.transpose(Mpart)


def kernel(x, edge_index, edge_attr, batch, W_sim1, b_sim1, W_lin1, b_lin1,
           W_e1a, b_e1a, W_e1b, b_e1b, W_sim2, b_sim2, W_lin2, b_lin2,
           W_e2a, b_e2a, W_e2b, b_e2b, W_fc, b_fc):
    src = edge_index[0]
    dst = edge_index[1]
    n = x.shape[0]
    deg = jnp.zeros((n,), jnp.float32).at[src].add(1.0)
    a_c = jax.nn.softplus(jnp.float32(1.0))
    w_e = a_c * edge_attr ** a_c

    npad = _MROWS - _NROWS
    ipad = jnp.zeros((npad, _GROW), jnp.int32)
    fpad = jnp.full((npad, _GROW), -1.0, jnp.float32)
    src2p = jnp.concatenate([src.reshape(_NROWS, _GROW), ipad], axis=0)
    dst2p = jnp.concatenate([dst.reshape(_NROWS, _GROW), ipad], axis=0)
    w2p = jnp.concatenate([w_e.reshape(_NROWS, _GROW), fpad], axis=0)
    edge2 = (src2p, dst2p, w2p, deg)

    xl1, Mcol1 = _layer(x, src, dst, edge2, W_sim1, b_sim1, W_lin1, b_lin1,
                        W_e1a, b_e1a, W_e1b, b_e1b, _IDX0_1)
    xl2, Mcol2 = _layer((xl1, Mcol1), src, dst, edge2, W_sim2, b_sim2,
                        W_lin2, b_lin2, W_e2a, b_e2a, W_e2b, b_e2b, _IDX0_2)
    return _final(xl2, Mcol2, batch, W_fc, b_fc)
